# speedup vs baseline: 6.3332x; 6.3332x over previous
import functools

import jax
import jax.numpy as jnp
from jax import lax
from jax.experimental import pallas as pl
from jax.experimental.pallas import tpu as pltpu
from jax.experimental.pallas import tpu_sc as plsc

_D = 128
_NW = 32
_CHUNK = 128


def _emb_body(nchunk, table_hbm, idx_hbm, out_hbm, idx_v, buf, sem):
    c = lax.axis_index("c")
    s = lax.axis_index("s")
    wid = s * 2 + c
    base = wid * (nchunk * _CHUNK)

    pltpu.sync_copy(idx_hbm.at[wid], idx_v)

    @pl.loop(0, nchunk)
    def _(j):
        pltpu.async_copy(table_hbm.at[idx_v.at[j]], buf, sem).wait()
        pltpu.sync_copy(buf, out_hbm.at[pl.ds(base + j * _CHUNK, _CHUNK)])


def kernel(key, table):
    b, h = key.shape
    total = b * h
    per_w = total // _NW
    nchunk = per_w // _CHUNK
    idx = key.reshape(_NW, nchunk, _CHUNK).astype(jnp.int32)

    mesh = plsc.VectorSubcoreMesh(core_axis_name="c", subcore_axis_name="s")
    out = pl.kernel(
        functools.partial(_emb_body, nchunk),
        out_type=jax.ShapeDtypeStruct((total, _D), jnp.float32),
        mesh=mesh,
        scratch_types=[
            pltpu.VMEM((nchunk, _CHUNK), jnp.int32),
            pltpu.VMEM((_CHUNK, _D), jnp.float32),
            pltpu.SemaphoreType.DMA,
        ],
    )(table, idx)
    return out.reshape(b, h, _D)


# speedup vs baseline: 9.1583x; 1.4461x over previous
import functools

import jax
import jax.numpy as jnp
from jax import lax
from jax.experimental import pallas as pl
from jax.experimental.pallas import tpu as pltpu
from jax.experimental.pallas import tpu_sc as plsc

_D = 128
_NW = 32
_CHUNK = 128


_NBUF = 4


def _emb_body(nchunk, table_hbm, idx_hbm, out_hbm, idx_v, buf, gsems, wsems):
    c = lax.axis_index("c")
    s = lax.axis_index("s")
    wid = s * 2 + c
    base = wid * (nchunk * _CHUNK)

    pltpu.sync_copy(idx_hbm.at[wid], idx_v)

    def gather(j, b):
        pltpu.async_copy(table_hbm.at[idx_v.at[j]], buf.at[b], gsems.at[b])

    def write(j, b):
        pltpu.async_copy(
            buf.at[b], out_hbm.at[pl.ds(base + j * _CHUNK, _CHUNK)], wsems.at[b]
        )

    def wait_gather(b):
        pltpu.make_async_copy(table_hbm.at[idx_v.at[0]], buf.at[b], gsems.at[b]).wait()

    def wait_write(b):
        pltpu.make_async_copy(
            buf.at[b], out_hbm.at[pl.ds(base, _CHUNK)], wsems.at[b]
        ).wait()

    for b in range(_NBUF):
        gather(b, b)

    nblk = nchunk // _NBUF

    @pl.loop(0, nblk - 1)
    def _(i):
        g = i * _NBUF
        for b in range(_NBUF):
            wait_gather(b)
            write(g + b, b)
        for b in range(_NBUF):
            wait_write(b)
            gather(g + _NBUF + b, b)

    g = (nblk - 1) * _NBUF
    for b in range(_NBUF):
        wait_gather(b)
        write(g + b, b)
    for b in range(_NBUF):
        wait_write(b)


def kernel(key, table):
    b, h = key.shape
    total = b * h
    per_w = total // _NW
    nchunk = per_w // _CHUNK
    idx = key.reshape(_NW, nchunk, _CHUNK).astype(jnp.int32)

    mesh = plsc.VectorSubcoreMesh(core_axis_name="c", subcore_axis_name="s")
    out = pl.kernel(
        functools.partial(_emb_body, nchunk),
        out_type=jax.ShapeDtypeStruct((total, _D), jnp.float32),
        mesh=mesh,
        scratch_types=[
            pltpu.VMEM((nchunk, _CHUNK), jnp.int32),
            pltpu.VMEM((_NBUF, _CHUNK, _D), jnp.float32),
            pltpu.SemaphoreType.DMA((_NBUF,)),
            pltpu.SemaphoreType.DMA((_NBUF,)),
        ],
    )(table, idx)
    return out.reshape(b, h, _D)
